# baseline (speedup 1.0000x reference)
import functools

import jax
import jax.numpy as jnp
from jax import lax
from jax.experimental import pallas as pl
from jax.experimental.pallas import tpu as pltpu
from jax.experimental.pallas import tpu_sc as plsc

N = 10000
E = 320000
DH = 128
NP = 10240
RB = 1024
NC = 2
NS = 16
NBUF = 4
LOOKAHEAD = 2
WB = NP // NS
ZR = 128

K_DEG = 100
T_DEG = E // K_DEG // (NC * NS)
K_F = 40
T_F = E // K_F // (NC * NS)
K_H = 125
T_H = E // K_H // NS

_MESH = plsc.VectorSubcoreMesh(
    core_axis_name="c", subcore_axis_name="s", num_cores=NC, num_subcores=NS)


def _zero_fill(ref, rows, cols):
  zero = jnp.zeros((16,), jnp.float32)
  per_row = cols // 16

  def body(r, carry):
    for c in range(per_row):
      ref[r, pl.ds(c * 16, 16)] = zero
    return carry

  lax.fori_loop(0, rows, body, 0)


def _zero_fill_1d(ref, n):
  zero = jnp.zeros((16,), jnp.float32)

  def body(i, carry):
    ref[pl.ds(i * 16, 16)] = zero
    return carry

  lax.fori_loop(0, n // 16, body, 0)


def _deg_body(didx_hbm, deg_out, didx_v, ones_v, zb_v, dacc, ssems):
  c = lax.axis_index("c")
  t = lax.axis_index("s")
  ld = pltpu.async_copy(didx_hbm.at[c * NS + t], didx_v, ssems.at[0])

  ones = jnp.ones((16,), jnp.float32)
  for i in range(8):
    ones_v[pl.ds(i * 16, 16)] = ones
  _zero_fill_1d(zb_v, WB)
  pltpu.sync_copy(zb_v, dacc.at[pl.ds(t * WB, WB)])
  ld.wait()
  plsc.subcore_barrier()

  src1 = ones_v.at[pl.ds(0, K_DEG)]

  def start(i, b):
    pltpu.async_copy(src1, dacc.at[didx_v.at[i]], ssems.at[b], add=True)

  def drain(i, b):
    pltpu.make_async_copy(src1, dacc.at[didx_v.at[i]], ssems.at[b]).wait()

  for db in range(NBUF):
    start(db, db)

  def outer(o, carry):
    for db in range(NBUF):
      i = o * NBUF + db
      drain(i - NBUF, db)
      start(i, db)
    return carry

  lax.fori_loop(1, T_DEG // NBUF, outer, 0)
  for db in range(NBUF):
    drain(T_DEG - NBUF + db, db)

  plsc.subcore_barrier()
  pltpu.sync_copy(dacc.at[pl.ds(t * WB, WB)],
                  deg_out.at[pl.ds(c * NP + t * WB, WB)])


_deg_kernel = pl.kernel(
    _deg_body,
    out_type=jax.ShapeDtypeStruct((NC * NP,), jnp.float32),
    mesh=_MESH,
    scratch_types=[
        pltpu.VMEM((T_DEG, K_DEG), jnp.int32),
        pltpu.VMEM((128,), jnp.float32),
        pltpu.VMEM((WB,), jnp.float32),
        pltpu.VMEM_SHARED((NP,), jnp.float32),
        pltpu.SemaphoreType.DMA((NBUF,)),
    ],
)


def _spmm_body(cols, k, tpr, edge_split, nbuf, la,
               table_hbm, sidx_hbm, didx_hbm, out_hbm,
               sidx_v, didx_v, gbuf, acc, gsems, ssems, zb_v=None):
  c = lax.axis_index("c")
  t = lax.axis_index("s")
  widx = c * NS + t if edge_split else t
  ld_s = pltpu.async_copy(sidx_hbm.at[widx], sidx_v, gsems.at[0])
  ld_d = pltpu.async_copy(didx_hbm.at[widx], didx_v, gsems.at[1])
  tab = table_hbm if edge_split else table_hbm.at[c]

  if edge_split:
    def idx_at(ref, i):
      return ref.at[pl.ds(pl.multiple_of(i * k, 8), k)]
  else:
    def idx_at(ref, i):
      return ref.at[i]

  if edge_split:
    _zero_fill(gbuf, ZR, cols)
    zsrc = gbuf.at[pl.ds(0, ZR)]
  else:
    _zero_fill(zb_v, ZR, cols)
    zsrc = zb_v
  wb0 = t * WB
  for z in range(WB // ZR):
    pltpu.async_copy(zsrc, acc.at[pl.ds(wb0 + z * ZR, ZR)],
                     ssems.at[z % nbuf])
  for z in range(WB // ZR):
    pltpu.make_async_copy(zsrc, acc.at[pl.ds(wb0 + z * ZR, ZR)],
                          ssems.at[z % nbuf]).wait()
  ld_s.wait()
  ld_d.wait()
  plsc.subcore_barrier()

  def gb(b):
    return gbuf.at[pl.ds(b * k, k)] if edge_split else gbuf.at[b]

  def start_gather(i, b):
    pltpu.async_copy(tab.at[idx_at(sidx_v, i)], gb(b), gsems.at[b])

  def wait_gather(i, b):
    pltpu.make_async_copy(tab.at[idx_at(sidx_v, i)], gb(b),
                          gsems.at[b]).wait()

  def start_scatter(i, b):
    pltpu.async_copy(gb(b), acc.at[idx_at(didx_v, i)], ssems.at[b], add=True)

  def wait_scatter(i, b):
    pltpu.make_async_copy(gb(b), acc.at[idx_at(didx_v, i)],
                          ssems.at[b]).wait()

  def step(i, db, first, last):
    j = i + la
    bj = (db + la) % nbuf
    if not last:
      if not first:
        wait_scatter(j - nbuf, bj)
      start_gather(j, bj)
    wait_gather(i, db)
    start_scatter(i, db)

  for i in range(la):
    start_gather(i, i % nbuf)
  head = (nbuf - la) + (tpr % nbuf)
  for i in range(head):
    step(i, i % nbuf, first=(i < nbuf - la), last=False)

  def outer(o, carry):
    for d in range(nbuf):
      i = head + o * nbuf + d
      step(i, (head + d) % nbuf, first=False, last=False)
    return carry

  lax.fori_loop(0, (tpr - la - head) // nbuf, outer, 0)

  for i in range(tpr - la, tpr):
    step(i, i % nbuf, first=False, last=True)
  for i in range(tpr - nbuf, tpr):
    wait_scatter(i, i % nbuf)

  plsc.subcore_barrier()
  for z in range(WB // ZR):
    pltpu.async_copy(acc.at[pl.ds(wb0 + z * ZR, ZR)],
                     out_hbm.at[c, pl.ds(wb0 + z * ZR, ZR)],
                     gsems.at[z % nbuf])
  for z in range(WB // ZR):
    pltpu.make_async_copy(acc.at[pl.ds(wb0 + z * ZR, ZR)],
                          out_hbm.at[c, pl.ds(wb0 + z * ZR, ZR)],
                          gsems.at[z % nbuf]).wait()


@functools.cache
def _make_spmm(cols, k, tpr, edge_split, nbuf=NBUF, la=LOOKAHEAD):
  scratch = [
      pltpu.VMEM((tpr * k,) if edge_split else (tpr, k), jnp.int32),
      pltpu.VMEM((tpr * k,) if edge_split else (tpr, k), jnp.int32),
      pltpu.VMEM((nbuf * k, cols) if edge_split else (nbuf, k, cols),
                 jnp.float32),
      pltpu.VMEM_SHARED((NP, cols), jnp.float32),
      pltpu.SemaphoreType.DMA((nbuf,)),
      pltpu.SemaphoreType.DMA((nbuf,)),
  ]
  if not edge_split:
    scratch.append(pltpu.VMEM((ZR, cols), jnp.float32))
  return pl.kernel(
      functools.partial(_spmm_body, cols, k, tpr, edge_split, nbuf, la),
      out_type=jax.ShapeDtypeStruct((NC, NP, cols), jnp.float32),
      mesh=_MESH,
      scratch_types=scratch,
      compiler_params=(None if edge_split else
                       pltpu.CompilerParams(use_tc_tiling_on_sc=False)),
  )


def _rowscale(m, s8):
  r, c = m.shape
  return (m.reshape(r // 128, 128, c) * s8[:, :, None]).reshape(r, c)


def _tc1_body(x_ref, w_ref, deg_ref, u_ref, s_ref):
  d = deg_ref[0, 0] + deg_ref[1, 0]
  s8 = lax.rsqrt(jnp.maximum(d, 1.0))
  s_ref[0] = s8
  g = jnp.dot(x_ref[...], w_ref[...], preferred_element_type=jnp.float32)
  u_ref[...] = _rowscale(g, s8)


def _tc_mid_body(split_out, v_ref, s_ref, b_ref, w_ref, u_ref):
  s8 = s_ref[0]
  v = v_ref[0] + v_ref[1]
  h = jnp.maximum(_rowscale(v, s8) + b_ref[0], 0.0)
  u = jnp.dot(h, w_ref[...], preferred_element_type=jnp.float32)
  u = _rowscale(u, s8)
  if split_out:
    half = u.shape[1] // 2
    u_ref[0] = u[:, :half]
    u_ref[1] = u[:, half:]
  else:
    u_ref[...] = u


def _tc_out_body(v_ref, s_ref, b_ref, o_ref):
  v = jnp.concatenate([v_ref[0], v_ref[1]], axis=1)
  o_ref[...] = _rowscale(v, s_ref[0]) + b_ref[0]


def _tc1(xp, w0, deg4):
  return pl.pallas_call(
      _tc1_body,
      grid=(NP // RB,),
      in_specs=[
          pl.BlockSpec((RB, DH), lambda i: (i, 0)),
          pl.BlockSpec((DH, DH), lambda i: (0, 0)),
          pl.BlockSpec((NC, 1, 8, 128), lambda i: (0, i, 0, 0)),
      ],
      out_specs=[
          pl.BlockSpec((RB, DH), lambda i: (i, 0)),
          pl.BlockSpec((1, 8, 128), lambda i: (i, 0, 0)),
      ],
      out_shape=[
          jax.ShapeDtypeStruct((NP, DH), jnp.float32),
          jax.ShapeDtypeStruct((NP // RB, 8, 128), jnp.float32),
      ],
  )(xp, w0, deg4)


def _tc_mid(v, s, b, w, out_cols, split_out):
  if split_out:
    out_spec = pl.BlockSpec((NC, RB, out_cols // 2), lambda i: (0, i, 0))
    out_shape = jax.ShapeDtypeStruct((NC, NP, out_cols // 2), jnp.float32)
  else:
    out_spec = pl.BlockSpec((RB, out_cols), lambda i: (i, 0))
    out_shape = jax.ShapeDtypeStruct((NP, out_cols), jnp.float32)
  return pl.pallas_call(
      functools.partial(_tc_mid_body, split_out),
      grid=(NP // RB,),
      in_specs=[
          pl.BlockSpec((NC, RB, DH), lambda i: (0, i, 0)),
          pl.BlockSpec((1, 8, 128), lambda i: (i, 0, 0)),
          pl.BlockSpec((1, DH), lambda i: (0, 0)),
          pl.BlockSpec((DH, out_cols), lambda i: (0, 0)),
      ],
      out_specs=out_spec,
      out_shape=out_shape,
  )(v, s, b, w)


def _tc_out(v, s, b):
  return pl.pallas_call(
      _tc_out_body,
      grid=(NP // RB,),
      in_specs=[
          pl.BlockSpec((NC, RB, 32), lambda i: (0, i, 0)),
          pl.BlockSpec((1, 8, 128), lambda i: (i, 0, 0)),
          pl.BlockSpec((1, 64), lambda i: (0, 0)),
      ],
      out_specs=pl.BlockSpec((RB, 64), lambda i: (i, 0)),
      out_shape=jax.ShapeDtypeStruct((NP, 64), jnp.float32),
  )(v, s, b)


def kernel(x, edge_index, W0, b0, W1, b1, W2, b2, full):
  del full
  src_f = edge_index[0].reshape(NC * NS, T_F * K_F)
  dst_f = edge_index[1].reshape(NC * NS, T_F * K_F)
  src_h = edge_index[0].reshape(NS, T_H, K_H)
  dst_h = edge_index[1].reshape(NS, T_H, K_H)
  dst_dg = edge_index[1].reshape(NC * NS, T_DEG, K_DEG)
  xp = jnp.pad(x, ((0, NP - N), (0, 0)))
  w2p = jnp.pad(W2, ((0, 0), (0, 64 - W2.shape[1])))
  b2p = jnp.pad(b2, (0, 64 - b2.shape[0])).reshape(1, 64)

  spmm_f = _make_spmm(DH, K_F, T_F, True, 5, 3)
  spmm_h = _make_spmm(32, K_H, T_H, False, 8, 4)

  deg = _deg_kernel(dst_dg)
  deg4 = deg.reshape(NC, NP // RB, 8, 128)
  u0, s = _tc1(xp, W0, deg4)
  v0 = spmm_f(u0, src_f, dst_f)
  u1 = _tc_mid(v0, s, b0.reshape(1, DH), W1, DH, False)
  v1 = spmm_f(u1, src_f, dst_f)
  u2 = _tc_mid(v1, s, b1.reshape(1, DH), w2p, 64, True)
  v2 = spmm_h(u2, src_h, dst_h)
  out = _tc_out(v2, s, b2p)
  return out[:N, :b2.shape[0]]


# speedup vs baseline: 1.0019x; 1.0019x over previous
import functools

import jax
import jax.numpy as jnp
from jax import lax
from jax.experimental import pallas as pl
from jax.experimental.pallas import tpu as pltpu
from jax.experimental.pallas import tpu_sc as plsc

N = 10000
E = 320000
DH = 128
NP = 10240
RB = 1024
NC = 2
NS = 16
NBUF = 4
LOOKAHEAD = 2
WB = NP // NS
ZR = 128

K_DEG = 100
T_DEG = E // K_DEG // (NC * NS)
K_F = 40
T_F = E // K_F // (NC * NS)
K_H = 125
T_H = E // K_H // NS

_MESH = plsc.VectorSubcoreMesh(
    core_axis_name="c", subcore_axis_name="s", num_cores=NC, num_subcores=NS)


def _zero_fill(ref, rows, cols):
  zero = jnp.zeros((16,), jnp.float32)
  per_row = cols // 16

  def body(r, carry):
    for c in range(per_row):
      ref[r, pl.ds(c * 16, 16)] = zero
    return carry

  lax.fori_loop(0, rows, body, 0)


def _zero_fill_1d(ref, n):
  zero = jnp.zeros((16,), jnp.float32)

  def body(i, carry):
    ref[pl.ds(i * 16, 16)] = zero
    return carry

  lax.fori_loop(0, n // 16, body, 0)


def _deg_body(didx_hbm, deg_out, didx_v, ones_v, zb_v, dacc, ssems):
  c = lax.axis_index("c")
  t = lax.axis_index("s")
  ld = pltpu.async_copy(didx_hbm.at[c * NS + t], didx_v, ssems.at[0])

  ones = jnp.ones((16,), jnp.float32)
  for i in range(8):
    ones_v[pl.ds(i * 16, 16)] = ones
  _zero_fill_1d(zb_v, WB)
  pltpu.sync_copy(zb_v, dacc.at[pl.ds(t * WB, WB)])
  ld.wait()
  plsc.subcore_barrier()

  src1 = ones_v.at[pl.ds(0, K_DEG)]

  def start(i, b):
    pltpu.async_copy(src1, dacc.at[didx_v.at[i]], ssems.at[b], add=True)

  def drain(i, b):
    pltpu.make_async_copy(src1, dacc.at[didx_v.at[i]], ssems.at[b]).wait()

  for db in range(NBUF):
    start(db, db)

  def outer(o, carry):
    for db in range(NBUF):
      i = o * NBUF + db
      drain(i - NBUF, db)
      start(i, db)
    return carry

  lax.fori_loop(1, T_DEG // NBUF, outer, 0)
  for db in range(NBUF):
    drain(T_DEG - NBUF + db, db)

  plsc.subcore_barrier()
  pltpu.sync_copy(dacc.at[pl.ds(t * WB, WB)],
                  deg_out.at[pl.ds(c * NP + t * WB, WB)])


_deg_kernel = pl.kernel(
    _deg_body,
    out_type=jax.ShapeDtypeStruct((NC * NP,), jnp.float32),
    mesh=_MESH,
    scratch_types=[
        pltpu.VMEM((T_DEG, K_DEG), jnp.int32),
        pltpu.VMEM((128,), jnp.float32),
        pltpu.VMEM((WB,), jnp.float32),
        pltpu.VMEM_SHARED((NP,), jnp.float32),
        pltpu.SemaphoreType.DMA((NBUF,)),
    ],
)


def _spmm_body(cols, k, tpr, edge_split, nbuf, la,
               table_hbm, sidx_hbm, didx_hbm, out_hbm,
               sidx_v, didx_v, gbuf, acc, gsems, ssems, zb_v=None):
  c = lax.axis_index("c")
  t = lax.axis_index("s")
  widx = c * NS + t if edge_split else t
  ld_s = pltpu.async_copy(sidx_hbm.at[widx], sidx_v, gsems.at[0])
  ld_d = pltpu.async_copy(didx_hbm.at[widx], didx_v, gsems.at[1])
  tab = table_hbm if edge_split else table_hbm.at[c]

  if edge_split:
    def idx_at(ref, i):
      return ref.at[pl.ds(pl.multiple_of(i * k, 8), k)]
  else:
    def idx_at(ref, i):
      return ref.at[i]

  if edge_split:
    _zero_fill(gbuf, ZR, cols)
    zsrc = gbuf.at[pl.ds(0, ZR)]
  else:
    _zero_fill(zb_v, ZR, cols)
    zsrc = zb_v
  wb0 = t * WB
  for z in range(WB // ZR):
    pltpu.async_copy(zsrc, acc.at[pl.ds(wb0 + z * ZR, ZR)],
                     ssems.at[z % nbuf])
  for z in range(WB // ZR):
    pltpu.make_async_copy(zsrc, acc.at[pl.ds(wb0 + z * ZR, ZR)],
                          ssems.at[z % nbuf]).wait()
  ld_s.wait()
  ld_d.wait()
  plsc.subcore_barrier()

  def gb(b):
    return gbuf.at[pl.ds(b * k, k)] if edge_split else gbuf.at[b]

  def start_gather(i, b):
    pltpu.async_copy(tab.at[idx_at(sidx_v, i)], gb(b), gsems.at[b])

  def wait_gather(i, b):
    pltpu.make_async_copy(tab.at[idx_at(sidx_v, i)], gb(b),
                          gsems.at[b]).wait()

  def start_scatter(i, b):
    pltpu.async_copy(gb(b), acc.at[idx_at(didx_v, i)], ssems.at[b], add=True)

  def wait_scatter(i, b):
    pltpu.make_async_copy(gb(b), acc.at[idx_at(didx_v, i)],
                          ssems.at[b]).wait()

  def step(i, db, first, last):
    j = i + la
    bj = (db + la) % nbuf
    if not last:
      if not first:
        wait_scatter(j - nbuf, bj)
      start_gather(j, bj)
    wait_gather(i, db)
    start_scatter(i, db)

  for i in range(la):
    start_gather(i, i % nbuf)
  head = (nbuf - la) + (tpr % nbuf)
  for i in range(head):
    step(i, i % nbuf, first=(i < nbuf - la), last=False)

  def outer(o, carry):
    for d in range(nbuf):
      i = head + o * nbuf + d
      step(i, (head + d) % nbuf, first=False, last=False)
    return carry

  lax.fori_loop(0, (tpr - la - head) // nbuf, outer, 0)

  for i in range(tpr - la, tpr):
    step(i, i % nbuf, first=False, last=True)
  for i in range(tpr - nbuf, tpr):
    wait_scatter(i, i % nbuf)

  plsc.subcore_barrier()
  for z in range(WB // ZR):
    pltpu.async_copy(acc.at[pl.ds(wb0 + z * ZR, ZR)],
                     out_hbm.at[c, pl.ds(wb0 + z * ZR, ZR)],
                     gsems.at[z % nbuf])
  for z in range(WB // ZR):
    pltpu.make_async_copy(acc.at[pl.ds(wb0 + z * ZR, ZR)],
                          out_hbm.at[c, pl.ds(wb0 + z * ZR, ZR)],
                          gsems.at[z % nbuf]).wait()


@functools.cache
def _make_spmm(cols, k, tpr, edge_split, nbuf=NBUF, la=LOOKAHEAD):
  scratch = [
      pltpu.VMEM((tpr * k,) if edge_split else (tpr, k), jnp.int32),
      pltpu.VMEM((tpr * k,) if edge_split else (tpr, k), jnp.int32),
      pltpu.VMEM((nbuf * k, cols) if edge_split else (nbuf, k, cols),
                 jnp.float32),
      pltpu.VMEM_SHARED((NP, cols), jnp.float32),
      pltpu.SemaphoreType.DMA((nbuf,)),
      pltpu.SemaphoreType.DMA((nbuf,)),
  ]
  if not edge_split:
    scratch.append(pltpu.VMEM((ZR, cols), jnp.float32))
  return pl.kernel(
      functools.partial(_spmm_body, cols, k, tpr, edge_split, nbuf, la),
      out_type=jax.ShapeDtypeStruct((NC, NP, cols), jnp.float32),
      mesh=_MESH,
      scratch_types=scratch,
      compiler_params=(None if edge_split else
                       pltpu.CompilerParams(use_tc_tiling_on_sc=False)),
  )


def _rowscale(m, s8):
  r, c = m.shape
  return (m.reshape(r // 128, 128, c) * s8[:, :, None]).reshape(r, c)


def _tc1_body(x_ref, w_ref, deg_ref, u_ref, s_ref):
  d = deg_ref[0, 0] + deg_ref[1, 0]
  s8 = lax.rsqrt(jnp.maximum(d, 1.0))
  s_ref[0] = s8
  g = jnp.dot(x_ref[...], w_ref[...], preferred_element_type=jnp.float32)
  u_ref[...] = _rowscale(g, s8)


def _tc_mid_body(split_out, v_ref, s_ref, b_ref, w_ref, u_ref):
  s8 = s_ref[0]
  v = v_ref[0] + v_ref[1]
  h = jnp.maximum(_rowscale(v, s8) + b_ref[0], 0.0)
  u = jnp.dot(h, w_ref[...], preferred_element_type=jnp.float32)
  u = _rowscale(u, s8)
  if split_out:
    half = u.shape[1] // 2
    u_ref[0] = u[:, :half]
    u_ref[1] = u[:, half:]
  else:
    u_ref[...] = u


def _tc_out_body(v_ref, s_ref, b_ref, o_ref):
  v = jnp.concatenate([v_ref[0], v_ref[1]], axis=1)
  o_ref[...] = _rowscale(v, s_ref[0]) + b_ref[0]


def _tc1(xp, w0, deg4):
  return pl.pallas_call(
      _tc1_body,
      grid=(NP // RB,),
      in_specs=[
          pl.BlockSpec((RB, DH), lambda i: (i, 0)),
          pl.BlockSpec((DH, DH), lambda i: (0, 0)),
          pl.BlockSpec((NC, 1, 8, 128), lambda i: (0, i, 0, 0)),
      ],
      out_specs=[
          pl.BlockSpec((RB, DH), lambda i: (i, 0)),
          pl.BlockSpec((1, 8, 128), lambda i: (i, 0, 0)),
      ],
      out_shape=[
          jax.ShapeDtypeStruct((NP, DH), jnp.float32),
          jax.ShapeDtypeStruct((NP // RB, 8, 128), jnp.float32),
      ],
  )(xp, w0, deg4)


def _tc_mid(v, s, b, w, out_cols, split_out):
  if split_out:
    out_spec = pl.BlockSpec((NC, RB, out_cols // 2), lambda i: (0, i, 0))
    out_shape = jax.ShapeDtypeStruct((NC, NP, out_cols // 2), jnp.float32)
  else:
    out_spec = pl.BlockSpec((RB, out_cols), lambda i: (i, 0))
    out_shape = jax.ShapeDtypeStruct((NP, out_cols), jnp.float32)
  return pl.pallas_call(
      functools.partial(_tc_mid_body, split_out),
      grid=(NP // RB,),
      in_specs=[
          pl.BlockSpec((NC, RB, DH), lambda i: (0, i, 0)),
          pl.BlockSpec((1, 8, 128), lambda i: (i, 0, 0)),
          pl.BlockSpec((1, DH), lambda i: (0, 0)),
          pl.BlockSpec((DH, out_cols), lambda i: (0, 0)),
      ],
      out_specs=out_spec,
      out_shape=out_shape,
  )(v, s, b, w)


def _tc_out(v, s, b):
  return pl.pallas_call(
      _tc_out_body,
      grid=(NP // RB,),
      in_specs=[
          pl.BlockSpec((NC, RB, 32), lambda i: (0, i, 0)),
          pl.BlockSpec((1, 8, 128), lambda i: (i, 0, 0)),
          pl.BlockSpec((1, 64), lambda i: (0, 0)),
      ],
      out_specs=pl.BlockSpec((RB, 64), lambda i: (i, 0)),
      out_shape=jax.ShapeDtypeStruct((NP, 64), jnp.float32),
  )(v, s, b)


def kernel(x, edge_index, W0, b0, W1, b1, W2, b2, full):
  del full
  src_f = edge_index[0].reshape(NC * NS, T_F * K_F)
  dst_f = edge_index[1].reshape(NC * NS, T_F * K_F)
  src_h = edge_index[0].reshape(NS, T_H, K_H)
  dst_h = edge_index[1].reshape(NS, T_H, K_H)
  dst_dg = edge_index[1].reshape(NC * NS, T_DEG, K_DEG)
  xp = jnp.pad(x, ((0, NP - N), (0, 0)))
  w2p = jnp.pad(W2, ((0, 0), (0, 64 - W2.shape[1])))
  b2p = jnp.pad(b2, (0, 64 - b2.shape[0])).reshape(1, 64)

  spmm_f = _make_spmm(DH, K_F, T_F, True, 5, 3)
  spmm_h = _make_spmm(32, K_H, T_H, False, 6, 3)

  deg = _deg_kernel(dst_dg)
  deg4 = deg.reshape(NC, NP // RB, 8, 128)
  u0, s = _tc1(xp, W0, deg4)
  v0 = spmm_f(u0, src_f, dst_f)
  u1 = _tc_mid(v0, s, b0.reshape(1, DH), W1, DH, False)
  v1 = spmm_f(u1, src_f, dst_f)
  u2 = _tc_mid(v1, s, b1.reshape(1, DH), w2p, 64, True)
  v2 = spmm_h(u2, src_h, dst_h)
  out = _tc_out(v2, s, b2p)
  return out[:N, :b2.shape[0]]


# speedup vs baseline: 1.0121x; 1.0101x over previous
import functools

import jax
import jax.numpy as jnp
from jax import lax
from jax.experimental import pallas as pl
from jax.experimental.pallas import tpu as pltpu
from jax.experimental.pallas import tpu_sc as plsc

N = 10000
E = 320000
DH = 128
NP = 10240
RB = 1024
NC = 2
NS = 16
NBUF = 4
LOOKAHEAD = 2
WB = NP // NS
ZR = 128

K_DEG = 40
T_DEG = E // K_DEG // (NC * NS)
K_F = 40
T_F = E // K_F // (NC * NS)
K_H = 125
T_H = E // K_H // NS

_MESH = plsc.VectorSubcoreMesh(
    core_axis_name="c", subcore_axis_name="s", num_cores=NC, num_subcores=NS)


def _zero_fill(ref, rows, cols):
  zero = jnp.zeros((16,), jnp.float32)
  per_row = cols // 16

  def body(r, carry):
    for c in range(per_row):
      ref[r, pl.ds(c * 16, 16)] = zero
    return carry

  lax.fori_loop(0, rows, body, 0)


def _zero_fill_1d(ref, n):
  zero = jnp.zeros((16,), jnp.float32)

  def body(i, carry):
    ref[pl.ds(i * 16, 16)] = zero
    return carry

  lax.fori_loop(0, n // 16, body, 0)


def _deg_body(didx_hbm, deg_out, didx_v, ones_v, zb_v, dacc, ssems):
  c = lax.axis_index("c")
  t = lax.axis_index("s")
  ld = pltpu.async_copy(didx_hbm.at[c * NS + t], didx_v, ssems.at[0])

  def idx_at(i):
    return didx_v.at[pl.ds(pl.multiple_of(i * K_DEG, 8), K_DEG)]

  ones = jnp.ones((16,), jnp.float32)
  for i in range(8):
    ones_v[pl.ds(i * 16, 16)] = ones
  _zero_fill_1d(zb_v, WB)
  pltpu.sync_copy(zb_v, dacc.at[pl.ds(t * WB, WB)])
  ld.wait()
  plsc.subcore_barrier()

  src1 = ones_v.at[pl.ds(0, K_DEG)]

  def start(i, b):
    pltpu.async_copy(src1, dacc.at[idx_at(i)], ssems.at[b], add=True)

  def drain(i, b):
    pltpu.make_async_copy(src1, dacc.at[idx_at(i)], ssems.at[b]).wait()

  for db in range(NBUF):
    start(db, db)

  def outer(o, carry):
    for db in range(NBUF):
      i = o * NBUF + db
      drain(i - NBUF, db)
      start(i, db)
    return carry

  lax.fori_loop(1, T_DEG // NBUF, outer, 0)
  for db in range(NBUF):
    drain(T_DEG - NBUF + db, db)

  plsc.subcore_barrier()
  pltpu.sync_copy(dacc.at[pl.ds(t * WB, WB)],
                  deg_out.at[pl.ds(c * NP + t * WB, WB)])


_deg_kernel = pl.kernel(
    _deg_body,
    out_type=jax.ShapeDtypeStruct((NC * NP,), jnp.float32),
    mesh=_MESH,
    scratch_types=[
        pltpu.VMEM((T_DEG * K_DEG,), jnp.int32),
        pltpu.VMEM((128,), jnp.float32),
        pltpu.VMEM((WB,), jnp.float32),
        pltpu.VMEM_SHARED((NP,), jnp.float32),
        pltpu.SemaphoreType.DMA((NBUF,)),
    ],
)


def _spmm_body(cols, k, tpr, edge_split, nbuf, la,
               table_hbm, sidx_hbm, didx_hbm, out_hbm,
               sidx_v, didx_v, gbuf, acc, gsems, ssems, zb_v=None):
  c = lax.axis_index("c")
  t = lax.axis_index("s")
  widx = c * NS + t if edge_split else t
  ld_s = pltpu.async_copy(sidx_hbm.at[widx], sidx_v, gsems.at[0])
  ld_d = pltpu.async_copy(didx_hbm.at[widx], didx_v, gsems.at[1])
  tab = table_hbm if edge_split else table_hbm.at[c]

  if edge_split:
    def idx_at(ref, i):
      return ref.at[pl.ds(pl.multiple_of(i * k, 8), k)]
  else:
    def idx_at(ref, i):
      return ref.at[i]

  if edge_split:
    _zero_fill(gbuf, ZR, cols)
    zsrc = gbuf.at[pl.ds(0, ZR)]
  else:
    _zero_fill(zb_v, ZR, cols)
    zsrc = zb_v
  wb0 = t * WB
  for z in range(WB // ZR):
    pltpu.async_copy(zsrc, acc.at[pl.ds(wb0 + z * ZR, ZR)],
                     ssems.at[z % nbuf])
  for z in range(WB // ZR):
    pltpu.make_async_copy(zsrc, acc.at[pl.ds(wb0 + z * ZR, ZR)],
                          ssems.at[z % nbuf]).wait()
  ld_s.wait()
  ld_d.wait()
  plsc.subcore_barrier()

  def gb(b):
    return gbuf.at[pl.ds(b * k, k)] if edge_split else gbuf.at[b]

  def start_gather(i, b):
    pltpu.async_copy(tab.at[idx_at(sidx_v, i)], gb(b), gsems.at[b])

  def wait_gather(i, b):
    pltpu.make_async_copy(tab.at[idx_at(sidx_v, i)], gb(b),
                          gsems.at[b]).wait()

  def start_scatter(i, b):
    pltpu.async_copy(gb(b), acc.at[idx_at(didx_v, i)], ssems.at[b], add=True)

  def wait_scatter(i, b):
    pltpu.make_async_copy(gb(b), acc.at[idx_at(didx_v, i)],
                          ssems.at[b]).wait()

  def step(i, db, first, last):
    j = i + la
    bj = (db + la) % nbuf
    if not last:
      if not first:
        wait_scatter(j - nbuf, bj)
      start_gather(j, bj)
    wait_gather(i, db)
    start_scatter(i, db)

  for i in range(la):
    start_gather(i, i % nbuf)
  head = (nbuf - la) + (tpr % nbuf)
  for i in range(head):
    step(i, i % nbuf, first=(i < nbuf - la), last=False)

  def outer(o, carry):
    for d in range(nbuf):
      i = head + o * nbuf + d
      step(i, (head + d) % nbuf, first=False, last=False)
    return carry

  lax.fori_loop(0, (tpr - la - head) // nbuf, outer, 0)

  for i in range(tpr - la, tpr):
    step(i, i % nbuf, first=False, last=True)
  for i in range(tpr - nbuf, tpr):
    wait_scatter(i, i % nbuf)

  plsc.subcore_barrier()
  for z in range(WB // ZR):
    pltpu.async_copy(acc.at[pl.ds(wb0 + z * ZR, ZR)],
                     out_hbm.at[c, pl.ds(wb0 + z * ZR, ZR)],
                     gsems.at[z % nbuf])
  for z in range(WB // ZR):
    pltpu.make_async_copy(acc.at[pl.ds(wb0 + z * ZR, ZR)],
                          out_hbm.at[c, pl.ds(wb0 + z * ZR, ZR)],
                          gsems.at[z % nbuf]).wait()


@functools.cache
def _make_spmm(cols, k, tpr, edge_split, nbuf=NBUF, la=LOOKAHEAD):
  scratch = [
      pltpu.VMEM((tpr * k,) if edge_split else (tpr, k), jnp.int32),
      pltpu.VMEM((tpr * k,) if edge_split else (tpr, k), jnp.int32),
      pltpu.VMEM((nbuf * k, cols) if edge_split else (nbuf, k, cols),
                 jnp.float32),
      pltpu.VMEM_SHARED((NP, cols), jnp.float32),
      pltpu.SemaphoreType.DMA((nbuf,)),
      pltpu.SemaphoreType.DMA((nbuf,)),
  ]
  if not edge_split:
    scratch.append(pltpu.VMEM((ZR, cols), jnp.float32))
  return pl.kernel(
      functools.partial(_spmm_body, cols, k, tpr, edge_split, nbuf, la),
      out_type=jax.ShapeDtypeStruct((NC, NP, cols), jnp.float32),
      mesh=_MESH,
      scratch_types=scratch,
      compiler_params=(None if edge_split else
                       pltpu.CompilerParams(use_tc_tiling_on_sc=False)),
  )


def _rowscale(m, s8):
  r, c = m.shape
  return (m.reshape(r // 128, 128, c) * s8[:, :, None]).reshape(r, c)


def _tc1_body(x_ref, w_ref, deg_ref, u_ref, s_ref):
  d = deg_ref[0, 0] + deg_ref[1, 0]
  s8 = lax.rsqrt(jnp.maximum(d, 1.0))
  s_ref[0] = s8
  g = jnp.dot(x_ref[...], w_ref[...], preferred_element_type=jnp.float32)
  u_ref[...] = _rowscale(g, s8)


def _tc_mid_body(split_out, v_ref, s_ref, b_ref, w_ref, u_ref):
  s8 = s_ref[0]
  v = v_ref[0] + v_ref[1]
  h = jnp.maximum(_rowscale(v, s8) + b_ref[0], 0.0)
  u = jnp.dot(h, w_ref[...], preferred_element_type=jnp.float32)
  u = _rowscale(u, s8)
  if split_out:
    half = u.shape[1] // 2
    u_ref[0] = u[:, :half]
    u_ref[1] = u[:, half:]
  else:
    u_ref[...] = u


def _tc_out_body(v_ref, s_ref, b_ref, o_ref):
  v = jnp.concatenate([v_ref[0], v_ref[1]], axis=1)
  o_ref[...] = _rowscale(v, s_ref[0]) + b_ref[0]


def _tc1(xp, w0, deg4):
  return pl.pallas_call(
      _tc1_body,
      grid=(NP // RB,),
      in_specs=[
          pl.BlockSpec((RB, DH), lambda i: (i, 0)),
          pl.BlockSpec((DH, DH), lambda i: (0, 0)),
          pl.BlockSpec((NC, 1, 8, 128), lambda i: (0, i, 0, 0)),
      ],
      out_specs=[
          pl.BlockSpec((RB, DH), lambda i: (i, 0)),
          pl.BlockSpec((1, 8, 128), lambda i: (i, 0, 0)),
      ],
      out_shape=[
          jax.ShapeDtypeStruct((NP, DH), jnp.float32),
          jax.ShapeDtypeStruct((NP // RB, 8, 128), jnp.float32),
      ],
  )(xp, w0, deg4)


def _tc_mid(v, s, b, w, out_cols, split_out):
  if split_out:
    out_spec = pl.BlockSpec((NC, RB, out_cols // 2), lambda i: (0, i, 0))
    out_shape = jax.ShapeDtypeStruct((NC, NP, out_cols // 2), jnp.float32)
  else:
    out_spec = pl.BlockSpec((RB, out_cols), lambda i: (i, 0))
    out_shape = jax.ShapeDtypeStruct((NP, out_cols), jnp.float32)
  return pl.pallas_call(
      functools.partial(_tc_mid_body, split_out),
      grid=(NP // RB,),
      in_specs=[
          pl.BlockSpec((NC, RB, DH), lambda i: (0, i, 0)),
          pl.BlockSpec((1, 8, 128), lambda i: (i, 0, 0)),
          pl.BlockSpec((1, DH), lambda i: (0, 0)),
          pl.BlockSpec((DH, out_cols), lambda i: (0, 0)),
      ],
      out_specs=out_spec,
      out_shape=out_shape,
  )(v, s, b, w)


def _tc_out(v, s, b):
  return pl.pallas_call(
      _tc_out_body,
      grid=(NP // RB,),
      in_specs=[
          pl.BlockSpec((NC, RB, 32), lambda i: (0, i, 0)),
          pl.BlockSpec((1, 8, 128), lambda i: (i, 0, 0)),
          pl.BlockSpec((1, 64), lambda i: (0, 0)),
      ],
      out_specs=pl.BlockSpec((RB, 64), lambda i: (i, 0)),
      out_shape=jax.ShapeDtypeStruct((NP, 64), jnp.float32),
  )(v, s, b)


def kernel(x, edge_index, W0, b0, W1, b1, W2, b2, full):
  del full
  src_f = edge_index[0].reshape(NC * NS, T_F * K_F)
  dst_f = edge_index[1].reshape(NC * NS, T_F * K_F)
  src_h = edge_index[0].reshape(NS, T_H, K_H)
  dst_h = edge_index[1].reshape(NS, T_H, K_H)

  xp = jnp.pad(x, ((0, NP - N), (0, 0)))
  w2p = jnp.pad(W2, ((0, 0), (0, 64 - W2.shape[1])))
  b2p = jnp.pad(b2, (0, 64 - b2.shape[0])).reshape(1, 64)

  spmm_f = _make_spmm(DH, K_F, T_F, True, 5, 3)
  spmm_h = _make_spmm(32, K_H, T_H, False, 6, 3)

  deg = _deg_kernel(dst_f)
  deg4 = deg.reshape(NC, NP // RB, 8, 128)
  u0, s = _tc1(xp, W0, deg4)
  v0 = spmm_f(u0, src_f, dst_f)
  u1 = _tc_mid(v0, s, b0.reshape(1, DH), W1, DH, False)
  v1 = spmm_f(u1, src_f, dst_f)
  u2 = _tc_mid(v1, s, b1.reshape(1, DH), w2p, 64, True)
  v2 = spmm_h(u2, src_h, dst_h)
  out = _tc_out(v2, s, b2p)
  return out[:N, :b2.shape[0]]


# speedup vs baseline: 1.0139x; 1.0018x over previous
import functools

import jax
import jax.numpy as jnp
from jax import lax
from jax.experimental import pallas as pl
from jax.experimental.pallas import tpu as pltpu
from jax.experimental.pallas import tpu_sc as plsc

N = 10000
E = 320000
DH = 128
NP = 10240
RB = 1024
NC = 2
NS = 16
NBUF = 4
LOOKAHEAD = 2
WB = NP // NS
ZR = 128

K_DEG = 40
T_DEG = E // K_DEG // (NC * NS)
DEG_NBUF = 5
assert T_DEG % DEG_NBUF == 0
K_F = 40
T_F = E // K_F // (NC * NS)
K_H = 125
T_H = E // K_H // NS

_MESH = plsc.VectorSubcoreMesh(
    core_axis_name="c", subcore_axis_name="s", num_cores=NC, num_subcores=NS)


def _zero_fill(ref, rows, cols):
  zero = jnp.zeros((16,), jnp.float32)
  per_row = cols // 16

  def body(r, carry):
    for c in range(per_row):
      ref[r, pl.ds(c * 16, 16)] = zero
    return carry

  lax.fori_loop(0, rows, body, 0)


def _zero_fill_1d(ref, n):
  zero = jnp.zeros((16,), jnp.float32)

  def body(i, carry):
    ref[pl.ds(i * 16, 16)] = zero
    return carry

  lax.fori_loop(0, n // 16, body, 0)


def _deg_body(didx_hbm, deg_out, didx_v, ones_v, zb_v, dacc, ssems):
  c = lax.axis_index("c")
  t = lax.axis_index("s")
  ld = pltpu.async_copy(didx_hbm.at[c * NS + t], didx_v, ssems.at[0])

  def idx_at(i):
    return didx_v.at[pl.ds(pl.multiple_of(i * K_DEG, 8), K_DEG)]

  ones = jnp.ones((16,), jnp.float32)
  for i in range(8):
    ones_v[pl.ds(i * 16, 16)] = ones
  _zero_fill_1d(zb_v, WB)
  pltpu.sync_copy(zb_v, dacc.at[pl.ds(t * WB, WB)])
  ld.wait()
  plsc.subcore_barrier()

  src1 = ones_v.at[pl.ds(0, K_DEG)]

  def start(i, b):
    pltpu.async_copy(src1, dacc.at[idx_at(i)], ssems.at[b], add=True)

  def drain(i, b):
    pltpu.make_async_copy(src1, dacc.at[idx_at(i)], ssems.at[b]).wait()

  for db in range(DEG_NBUF):
    start(db, db)

  def outer(o, carry):
    for db in range(DEG_NBUF):
      i = o * DEG_NBUF + db
      drain(i - DEG_NBUF, db)
      start(i, db)
    return carry

  lax.fori_loop(1, T_DEG // DEG_NBUF, outer, 0)
  for db in range(DEG_NBUF):
    drain(T_DEG - DEG_NBUF + db, db)

  plsc.subcore_barrier()
  pltpu.sync_copy(dacc.at[pl.ds(t * WB, WB)],
                  deg_out.at[pl.ds(c * NP + t * WB, WB)])


_deg_kernel = pl.kernel(
    _deg_body,
    out_type=jax.ShapeDtypeStruct((NC * NP,), jnp.float32),
    mesh=_MESH,
    scratch_types=[
        pltpu.VMEM((T_DEG * K_DEG,), jnp.int32),
        pltpu.VMEM((128,), jnp.float32),
        pltpu.VMEM((WB,), jnp.float32),
        pltpu.VMEM_SHARED((NP,), jnp.float32),
        pltpu.SemaphoreType.DMA((DEG_NBUF,)),
    ],
)


def _spmm_body(cols, k, tpr, edge_split, nbuf, la,
               table_hbm, sidx_hbm, didx_hbm, out_hbm,
               sidx_v, didx_v, gbuf, acc, gsems, ssems, zb_v=None):
  c = lax.axis_index("c")
  t = lax.axis_index("s")
  widx = c * NS + t if edge_split else t
  ld_s = pltpu.async_copy(sidx_hbm.at[widx], sidx_v, gsems.at[0])
  ld_d = pltpu.async_copy(didx_hbm.at[widx], didx_v, gsems.at[1])
  tab = table_hbm if edge_split else table_hbm.at[c]

  if edge_split:
    def idx_at(ref, i):
      return ref.at[pl.ds(pl.multiple_of(i * k, 8), k)]
  else:
    def idx_at(ref, i):
      return ref.at[i]

  if edge_split:
    _zero_fill(gbuf, ZR, cols)
    zsrc = gbuf.at[pl.ds(0, ZR)]
  else:
    _zero_fill(zb_v, ZR, cols)
    zsrc = zb_v
  wb0 = t * WB
  for z in range(WB // ZR):
    pltpu.async_copy(zsrc, acc.at[pl.ds(wb0 + z * ZR, ZR)],
                     ssems.at[z % nbuf])
  for z in range(WB // ZR):
    pltpu.make_async_copy(zsrc, acc.at[pl.ds(wb0 + z * ZR, ZR)],
                          ssems.at[z % nbuf]).wait()
  ld_s.wait()
  ld_d.wait()
  plsc.subcore_barrier()

  def gb(b):
    return gbuf.at[pl.ds(b * k, k)] if edge_split else gbuf.at[b]

  def start_gather(i, b):
    pltpu.async_copy(tab.at[idx_at(sidx_v, i)], gb(b), gsems.at[b])

  def wait_gather(i, b):
    pltpu.make_async_copy(tab.at[idx_at(sidx_v, i)], gb(b),
                          gsems.at[b]).wait()

  def start_scatter(i, b):
    pltpu.async_copy(gb(b), acc.at[idx_at(didx_v, i)], ssems.at[b], add=True)

  def wait_scatter(i, b):
    pltpu.make_async_copy(gb(b), acc.at[idx_at(didx_v, i)],
                          ssems.at[b]).wait()

  def step(i, db, first, last):
    j = i + la
    bj = (db + la) % nbuf
    if not last:
      if not first:
        wait_scatter(j - nbuf, bj)
      start_gather(j, bj)
    wait_gather(i, db)
    start_scatter(i, db)

  for i in range(la):
    start_gather(i, i % nbuf)
  head = (nbuf - la) + (tpr % nbuf)
  for i in range(head):
    step(i, i % nbuf, first=(i < nbuf - la), last=False)

  def outer(o, carry):
    for d in range(nbuf):
      i = head + o * nbuf + d
      step(i, (head + d) % nbuf, first=False, last=False)
    return carry

  lax.fori_loop(0, (tpr - la - head) // nbuf, outer, 0)

  for i in range(tpr - la, tpr):
    step(i, i % nbuf, first=False, last=True)
  for i in range(tpr - nbuf, tpr):
    wait_scatter(i, i % nbuf)

  plsc.subcore_barrier()
  for z in range(WB // ZR):
    pltpu.async_copy(acc.at[pl.ds(wb0 + z * ZR, ZR)],
                     out_hbm.at[c, pl.ds(wb0 + z * ZR, ZR)],
                     gsems.at[z % nbuf])
  for z in range(WB // ZR):
    pltpu.make_async_copy(acc.at[pl.ds(wb0 + z * ZR, ZR)],
                          out_hbm.at[c, pl.ds(wb0 + z * ZR, ZR)],
                          gsems.at[z % nbuf]).wait()


@functools.cache
def _make_spmm(cols, k, tpr, edge_split, nbuf=NBUF, la=LOOKAHEAD):
  scratch = [
      pltpu.VMEM((tpr * k,) if edge_split else (tpr, k), jnp.int32),
      pltpu.VMEM((tpr * k,) if edge_split else (tpr, k), jnp.int32),
      pltpu.VMEM((nbuf * k, cols) if edge_split else (nbuf, k, cols),
                 jnp.float32),
      pltpu.VMEM_SHARED((NP, cols), jnp.float32),
      pltpu.SemaphoreType.DMA((nbuf,)),
      pltpu.SemaphoreType.DMA((nbuf,)),
  ]
  if not edge_split:
    scratch.append(pltpu.VMEM((ZR, cols), jnp.float32))
  return pl.kernel(
      functools.partial(_spmm_body, cols, k, tpr, edge_split, nbuf, la),
      out_type=jax.ShapeDtypeStruct((NC, NP, cols), jnp.float32),
      mesh=_MESH,
      scratch_types=scratch,
      compiler_params=(None if edge_split else
                       pltpu.CompilerParams(use_tc_tiling_on_sc=False)),
  )


def _rowscale(m, s8):
  r, c = m.shape
  return (m.reshape(r // 128, 128, c) * s8[:, :, None]).reshape(r, c)


def _tc1_body(x_ref, w_ref, deg_ref, u_ref, s_ref):
  d = deg_ref[0, 0] + deg_ref[1, 0]
  s8 = lax.rsqrt(jnp.maximum(d, 1.0))
  s_ref[0] = s8
  g = jnp.dot(x_ref[...], w_ref[...], preferred_element_type=jnp.float32)
  u_ref[...] = _rowscale(g, s8)


def _tc_mid_body(split_out, v_ref, s_ref, b_ref, w_ref, u_ref):
  s8 = s_ref[0]
  v = v_ref[0] + v_ref[1]
  h = jnp.maximum(_rowscale(v, s8) + b_ref[0], 0.0)
  u = jnp.dot(h, w_ref[...], preferred_element_type=jnp.float32)
  u = _rowscale(u, s8)
  if split_out:
    half = u.shape[1] // 2
    u_ref[0] = u[:, :half]
    u_ref[1] = u[:, half:]
  else:
    u_ref[...] = u


def _tc_out_body(v_ref, s_ref, b_ref, o_ref):
  v = jnp.concatenate([v_ref[0], v_ref[1]], axis=1)
  o_ref[...] = _rowscale(v, s_ref[0]) + b_ref[0]


def _tc1(xp, w0, deg4):
  return pl.pallas_call(
      _tc1_body,
      grid=(NP // RB,),
      in_specs=[
          pl.BlockSpec((RB, DH), lambda i: (i, 0)),
          pl.BlockSpec((DH, DH), lambda i: (0, 0)),
          pl.BlockSpec((NC, 1, 8, 128), lambda i: (0, i, 0, 0)),
      ],
      out_specs=[
          pl.BlockSpec((RB, DH), lambda i: (i, 0)),
          pl.BlockSpec((1, 8, 128), lambda i: (i, 0, 0)),
      ],
      out_shape=[
          jax.ShapeDtypeStruct((NP, DH), jnp.float32),
          jax.ShapeDtypeStruct((NP // RB, 8, 128), jnp.float32),
      ],
  )(xp, w0, deg4)


def _tc_mid(v, s, b, w, out_cols, split_out):
  if split_out:
    out_spec = pl.BlockSpec((NC, RB, out_cols // 2), lambda i: (0, i, 0))
    out_shape = jax.ShapeDtypeStruct((NC, NP, out_cols // 2), jnp.float32)
  else:
    out_spec = pl.BlockSpec((RB, out_cols), lambda i: (i, 0))
    out_shape = jax.ShapeDtypeStruct((NP, out_cols), jnp.float32)
  return pl.pallas_call(
      functools.partial(_tc_mid_body, split_out),
      grid=(NP // RB,),
      in_specs=[
          pl.BlockSpec((NC, RB, DH), lambda i: (0, i, 0)),
          pl.BlockSpec((1, 8, 128), lambda i: (i, 0, 0)),
          pl.BlockSpec((1, DH), lambda i: (0, 0)),
          pl.BlockSpec((DH, out_cols), lambda i: (0, 0)),
      ],
      out_specs=out_spec,
      out_shape=out_shape,
  )(v, s, b, w)


def _tc_out(v, s, b):
  return pl.pallas_call(
      _tc_out_body,
      grid=(NP // RB,),
      in_specs=[
          pl.BlockSpec((NC, RB, 32), lambda i: (0, i, 0)),
          pl.BlockSpec((1, 8, 128), lambda i: (i, 0, 0)),
          pl.BlockSpec((1, 64), lambda i: (0, 0)),
      ],
      out_specs=pl.BlockSpec((RB, 64), lambda i: (i, 0)),
      out_shape=jax.ShapeDtypeStruct((NP, 64), jnp.float32),
  )(v, s, b)


def kernel(x, edge_index, W0, b0, W1, b1, W2, b2, full):
  del full
  src_f = edge_index[0].reshape(NC * NS, T_F * K_F)
  dst_f = edge_index[1].reshape(NC * NS, T_F * K_F)
  src_h = edge_index[0].reshape(NS, T_H, K_H)
  dst_h = edge_index[1].reshape(NS, T_H, K_H)

  xp = jnp.pad(x, ((0, NP - N), (0, 0)))
  w2p = jnp.pad(W2, ((0, 0), (0, 64 - W2.shape[1])))
  b2p = jnp.pad(b2, (0, 64 - b2.shape[0])).reshape(1, 64)

  spmm_f = _make_spmm(DH, K_F, T_F, True, 5, 3)
  spmm_h = _make_spmm(32, K_H, T_H, False, 6, 3)

  deg = _deg_kernel(dst_f)
  deg4 = deg.reshape(NC, NP // RB, 8, 128)
  u0, s = _tc1(xp, W0, deg4)
  v0 = spmm_f(u0, src_f, dst_f)
  u1 = _tc_mid(v0, s, b0.reshape(1, DH), W1, DH, False)
  v1 = spmm_f(u1, src_f, dst_f)
  u2 = _tc_mid(v1, s, b1.reshape(1, DH), w2p, 64, True)
  v2 = spmm_h(u2, src_h, dst_h)
  out = _tc_out(v2, s, b2p)
  return out[:N, :b2.shape[0]]
